# baseline (speedup 1.0000x reference)
import functools

import jax
import jax.numpy as jnp
from jax import lax
from jax.experimental import pallas as pl
from jax.experimental.pallas import tpu as pltpu
from jax.experimental.pallas import tpu_sc as plsc

B, T, G = 16, 512, 128
H, GH = 768, 128
L = T + G + 3
LPAD = 672
HALF = LPAD // 2
KCH = 112
NCH = HALF // KCH
GHALF = G // 2


def _sc_pack(text_hbm, tag_hbm, uid_hbm, word_hbm, bert_hbm, gnn_hbm,
             user_hbm, base_hbm, bertc_hbm, gnnc_hbm, ue_hbm,
             tids_v, gids_v, wsrc_v, tcmp_v, wbuf0_v, gbuf_v,
             ubuf_v, uidx_v, sg0, sw0, sm):
    cid = lax.axis_index("c")
    sid = lax.axis_index("s")
    b = cid * 8 + sid // 2
    h = sid % 2
    lane = lax.iota(jnp.int32, 16)
    zero16 = jnp.zeros((16,), jnp.int32)

    pltpu.sync_copy(text_hbm.at[b], tids_v)
    pltpu.sync_copy(tag_hbm.at[b], gids_v)

    def zbody(i, c):
        wsrc_v[pl.ds(i * 16, 16)] = zero16
        return c
    lax.fori_loop(0, LPAD // 16, zbody, 0)

    def z2body(i, c):
        tcmp_v[pl.ds(i * 16, 16)] = zero16
        return c
    lax.fori_loop(0, G // 16, z2body, 0)

    def tbody(j, cnt):
        v = tids_v[pl.ds(j * 16, 16)]
        m = (v > 0) & ((lane + j * 16) > 0)
        mi = m.astype(jnp.int32)
        cs = plsc.cumsum(mi) + cnt
        plsc.store_scatter(wsrc_v, [cs + 2], v, mask=m)
        return cnt + jnp.sum(mi)
    text_cnt = lax.fori_loop(0, T // 16, tbody, jnp.int32(0))
    text_end = text_cnt + 3

    def gbody(j, cnt):
        v = gids_v[pl.ds(j * 16, 16)]
        m = v > 0
        mi = m.astype(jnp.int32)
        cs = plsc.cumsum(mi) + cnt
        plsc.store_scatter(tcmp_v, [cs - 1], v, mask=m)
        return cnt + jnp.sum(mi)
    tag_cnt = lax.fori_loop(0, G // 16, gbody, jnp.int32(0))
    tag_end = text_end + tag_cnt

    sep_col = jnp.where(text_cnt > 0, text_cnt, T - 1)
    sep_vec = plsc.load_gather(tids_v, [jnp.broadcast_to(sep_col, (16,))])
    cls_vec = plsc.load_gather(tids_v, [zero16])
    sp_idx = jnp.where(lane == 0, 0, jnp.where(lane == 1, 2, tag_end))
    sp_val = jnp.where(lane == 0, cls_vec, sep_vec)
    plsc.store_scatter(wsrc_v, [sp_idx], sp_val, mask=lane < 3)

    gg = pltpu.async_copy(gnn_hbm.at[tcmp_v.at[pl.ds(h * GHALF, GHALF)]], gbuf_v, sm)

    def g_start(j):
        c0 = h * HALF + j * KCH
        return pltpu.async_copy(
            word_hbm.at[wsrc_v.at[pl.ds(c0, KCH)]], wbuf0_v, sg0)

    def w_start(j):
        c0 = h * HALF + j * KCH
        return pltpu.async_copy(
            wbuf0_v, base_hbm.at[b, pl.ds(c0, KCH)], sw0)

    for j in range(NCH):
        g_start(j).wait()
        w_start(j).wait()

    bg0 = pltpu.async_copy(
        bert_hbm.at[tcmp_v.at[pl.ds(h * GHALF, GHALF)]], wbuf0_v.at[pl.ds(0, GHALF)], sg0)
    bg0.wait()
    bw0 = pltpu.async_copy(
        wbuf0_v.at[pl.ds(0, GHALF)], bertc_hbm.at[b, pl.ds(h * GHALF, GHALF)], sw0)
    gg.wait()
    gw = pltpu.async_copy(gbuf_v, gnnc_hbm.at[b, pl.ds(h * GHALF, GHALF)], sm)
    bw0.wait()
    gw.wait()

    @pl.when((cid == 0) & (sid == 0))
    def _():
        pltpu.sync_copy(uid_hbm, uidx_v)
        pltpu.async_copy(user_hbm.at[uidx_v], ubuf_v, sm).wait()
        pltpu.sync_copy(ubuf_v, ue_hbm)


@functools.cache
def _make_sc_call():
    return pl.kernel(
        _sc_pack,
        out_type=[
            jax.ShapeDtypeStruct((B, LPAD, H), jnp.float32),
            jax.ShapeDtypeStruct((B, G, H), jnp.float32),
            jax.ShapeDtypeStruct((B, G, GH), jnp.float32),
            jax.ShapeDtypeStruct((B, GH), jnp.float32),
        ],
        mesh=plsc.VectorSubcoreMesh(core_axis_name="c", subcore_axis_name="s"),
        compiler_params=pltpu.CompilerParams(needs_layout_passes=False),
        scratch_types=[
            pltpu.VMEM((T,), jnp.int32),
            pltpu.VMEM((G,), jnp.int32),
            pltpu.VMEM((LPAD,), jnp.int32),
            pltpu.VMEM((G,), jnp.int32),
            pltpu.VMEM((KCH, H), jnp.float32),
            pltpu.VMEM((GHALF, GH), jnp.float32),
            pltpu.VMEM((B, GH), jnp.float32),
            pltpu.VMEM((B,), jnp.int32),
            pltpu.SemaphoreType.DMA,
            pltpu.SemaphoreType.DMA,
            pltpu.SemaphoreType.DMA,
        ],
    )


def _tc_finish(text_ref, tag_ref, base_ref, bertc_ref, gnnc_ref, ue_ref,
               pos_ref, wtag_ref, btag_ref, wuser_ref, buser_ref, type_ref,
               lnw_ref, lnb_ref, out_ref, attn_ref, tz_ref, p0_ref, p1_ref):
    bidx = pl.program_id(0)
    trow = text_ref[0]
    grow = tag_ref[0]
    text_cnt = jnp.sum((trow[:, 1:] > 0).astype(jnp.int32))
    tag_cnt = jnp.sum((grow > 0).astype(jnp.int32))
    text_end = text_cnt + 3
    tag_end = text_end + tag_cnt
    typ = type_ref[...]

    @pl.when(bidx == 0)
    def _():
        pos_a0 = pos_ref[0:LPAD, :]
        p0_ref[...] = pos_a0 + typ[0:1]
        p1_ref[...] = pos_a0 + typ[1:2]
        tz_ref[pl.ds(G, LPAD - G), :] = jnp.zeros((LPAD - G, H), jnp.float32)

    ge = lax.dot_general(gnnc_ref[0], wtag_ref[...], (((1,), (1,)), ((), ())),
                         preferred_element_type=jnp.float32)
    tz_ref[pl.ds(0, G), :] = (ge + btag_ref[...]) * bertc_ref[0]
    emb = base_ref[0] + pltpu.roll(tz_ref[...], text_end, 0)

    shift = tag_cnt - 1
    shift = jnp.where(shift < 0, shift + LPAD, shift)
    zc = pltpu.roll(p0_ref[...], shift, 0)
    yrow = pos_ref[pl.ds(text_end, 1), :] + typ[2:3]
    g = lax.broadcasted_iota(jnp.int32, (LPAD, 1), 0)
    emb = emb + jnp.where(g < text_end, p1_ref[...],
                          jnp.where(g <= tag_end, yrow, zc))

    lnw = lnw_ref[...]
    lnb = lnb_ref[...]

    def _ln(x):
        mu = jnp.mean(x, axis=-1, keepdims=True)
        var = jnp.mean((x - mu) ** 2, axis=-1, keepdims=True)
        return (x - mu) * lax.rsqrt(var + 1e-12) * lnw + lnb

    out_ref[0] = _ln(emb)[:L]

    ue_row = ue_ref[pl.ds(bidx, 1), :]
    uevec = lax.dot_general(ue_row, wuser_ref[...], (((1,), (1,)), ((), ())),
                            preferred_element_type=jnp.float32) + buser_ref[...]
    g3 = lax.broadcasted_iota(jnp.int32, (3, 1), 0)
    fix3 = (base_ref[0, 0:3, :] + pos_ref[0:3, :] + typ[0:1]
            + jnp.where(g3 == 1, 1.0, 0.0) * uevec)
    out_ref[0, 0:3, :] = _ln(fix3)
    rowt = (base_ref[0, pl.ds(tag_end, 1), :]
            + pos_ref[pl.ds(text_end + 1, 1), :] + typ[2:3])
    out_ref[0, pl.ds(tag_end, 1), :] = _ln(rowt)

    attn_ref[0] = (lax.broadcasted_iota(jnp.int32, (1, L), 1)
                   <= tag_end).astype(jnp.int32)


def kernel(user_ids, text_ids, tag_ids, user_table, word_table, bert_tag_table,
           gnn_tag_table, pos_table, type_table, W_user, b_user, W_tag, b_tag,
           ln_w, ln_b):
    text_ids = text_ids.astype(jnp.int32)
    tag_ids = tag_ids.astype(jnp.int32)
    uid_flat = user_ids.reshape(B).astype(jnp.int32)

    base, bert_c, gnn_c, ue = _make_sc_call()(
        text_ids, tag_ids, uid_flat, word_table, bert_tag_table,
        gnn_tag_table, user_table)

    text3 = text_ids.reshape(B, 1, T)
    tag3 = tag_ids.reshape(B, 1, G)
    out, attn = pl.pallas_call(
        _tc_finish,
        grid=(B,),
        compiler_params=pltpu.CompilerParams(
            dimension_semantics=("parallel",)),
        in_specs=[
            pl.BlockSpec((1, 1, T), lambda i: (i, 0, 0)),
            pl.BlockSpec((1, 1, G), lambda i: (i, 0, 0)),
            pl.BlockSpec((1, LPAD, H), lambda i: (i, 0, 0)),
            pl.BlockSpec((1, G, H), lambda i: (i, 0, 0)),
            pl.BlockSpec((1, G, GH), lambda i: (i, 0, 0)),
            pl.BlockSpec((B, GH), lambda i: (0, 0)),
            pl.BlockSpec((1024, H), lambda i: (0, 0)),
            pl.BlockSpec((H, GH), lambda i: (0, 0)),
            pl.BlockSpec((1, H), lambda i: (0, 0)),
            pl.BlockSpec((H, GH), lambda i: (0, 0)),
            pl.BlockSpec((1, H), lambda i: (0, 0)),
            pl.BlockSpec((3, H), lambda i: (0, 0)),
            pl.BlockSpec((1, H), lambda i: (0, 0)),
            pl.BlockSpec((1, H), lambda i: (0, 0)),
        ],
        out_specs=[
            pl.BlockSpec((1, L, H), lambda i: (i, 0, 0)),
            pl.BlockSpec((1, 1, L), lambda i: (i, 0, 0)),
        ],
        out_shape=[
            jax.ShapeDtypeStruct((B, L, H), jnp.float32),
            jax.ShapeDtypeStruct((B, 1, L), jnp.int32),
        ],
        scratch_shapes=[
            pltpu.VMEM((LPAD, H), jnp.float32),
            pltpu.VMEM((LPAD, H), jnp.float32),
            pltpu.VMEM((LPAD, H), jnp.float32),
        ],
    )(text3, tag3, base, bert_c, gnn_c, ue, pos_table, W_tag,
      b_tag.reshape(1, H), W_user, b_user.reshape(1, H), type_table,
      ln_w.reshape(1, H), ln_b.reshape(1, H))
    return out, attn.reshape(B, L)


# speedup vs baseline: 1.0216x; 1.0216x over previous
import functools

import jax
import jax.numpy as jnp
from jax import lax
from jax.experimental import pallas as pl
from jax.experimental.pallas import tpu as pltpu
from jax.experimental.pallas import tpu_sc as plsc

B, T, G = 16, 512, 128
H, GH = 768, 128
L = T + G + 3
LPAD = 672
HALF = LPAD // 2
KCH = 48
NCH = HALF // KCH
GHALF = G // 2


def _sc_pack(text_hbm, tag_hbm, uid_hbm, word_hbm, bert_hbm, gnn_hbm,
             user_hbm, base_hbm, bertc_hbm, gnnc_hbm, ue_hbm,
             tids_v, gids_v, wsrc_v, tcmp_v, wbuf0_v, wbuf1_v, wbuf2_v, gbuf_v,
             ubuf_v, uidx_v, sg0, sg1, sg2, sw0, sw1, sw2, sm):
    cid = lax.axis_index("c")
    sid = lax.axis_index("s")
    b = cid * 8 + sid // 2
    h = sid % 2
    lane = lax.iota(jnp.int32, 16)
    zero16 = jnp.zeros((16,), jnp.int32)

    pltpu.sync_copy(text_hbm.at[b], tids_v)
    pltpu.sync_copy(tag_hbm.at[b], gids_v)

    def zbody(i, c):
        wsrc_v[pl.ds(i * 16, 16)] = zero16
        return c
    lax.fori_loop(0, LPAD // 16, zbody, 0)

    def z2body(i, c):
        tcmp_v[pl.ds(i * 16, 16)] = zero16
        return c
    lax.fori_loop(0, G // 16, z2body, 0)

    def tbody(j, cnt):
        v = tids_v[pl.ds(j * 16, 16)]
        m = (v > 0) & ((lane + j * 16) > 0)
        mi = m.astype(jnp.int32)
        cs = plsc.cumsum(mi) + cnt
        plsc.store_scatter(wsrc_v, [cs + 2], v, mask=m)
        return cnt + jnp.sum(mi)
    text_cnt = lax.fori_loop(0, T // 16, tbody, jnp.int32(0))
    text_end = text_cnt + 3

    def gbody(j, cnt):
        v = gids_v[pl.ds(j * 16, 16)]
        m = v > 0
        mi = m.astype(jnp.int32)
        cs = plsc.cumsum(mi) + cnt
        plsc.store_scatter(tcmp_v, [cs - 1], v, mask=m)
        return cnt + jnp.sum(mi)
    tag_cnt = lax.fori_loop(0, G // 16, gbody, jnp.int32(0))
    tag_end = text_end + tag_cnt

    sep_col = jnp.where(text_cnt > 0, text_cnt, T - 1)
    sep_vec = plsc.load_gather(tids_v, [jnp.broadcast_to(sep_col, (16,))])
    cls_vec = plsc.load_gather(tids_v, [zero16])
    sp_idx = jnp.where(lane == 0, 0, jnp.where(lane == 1, 2, tag_end))
    sp_val = jnp.where(lane == 0, cls_vec, sep_vec)
    plsc.store_scatter(wsrc_v, [sp_idx], sp_val, mask=lane < 3)

    bufs = (wbuf0_v, wbuf1_v, wbuf2_v)
    gsems = (sg0, sg1, sg2)
    wsems = (sw0, sw1, sw2)
    NBUF = 3

    def g_start(j):
        c0 = h * HALF + j * KCH
        return pltpu.async_copy(
            word_hbm.at[wsrc_v.at[pl.ds(c0, KCH)]], bufs[j % NBUF], gsems[j % NBUF])

    def w_start(j):
        c0 = h * HALF + j * KCH
        return pltpu.async_copy(
            bufs[j % NBUF], base_hbm.at[b, pl.ds(c0, KCH)], wsems[j % NBUF])

    gd = {j: g_start(j) for j in range(NBUF)}
    wd = {}
    for j in range(NCH):
        gd[j].wait()
        wd[j] = w_start(j)
        if j + NBUF < NCH:
            wd[j].wait()
            gd[j + NBUF] = g_start(j + NBUF)

    gg = pltpu.async_copy(gnn_hbm.at[tcmp_v.at[pl.ds(h * GHALF, GHALF)]], gbuf_v, sm)
    wd[NCH - 3].wait()
    wd[NCH - 2].wait()
    wd[NCH - 1].wait()
    bg0 = pltpu.async_copy(
        bert_hbm.at[tcmp_v.at[pl.ds(h * GHALF, 32)]], wbuf0_v.at[pl.ds(0, 32)], sg0)
    bg1 = pltpu.async_copy(
        bert_hbm.at[tcmp_v.at[pl.ds(h * GHALF + 32, 32)]], wbuf1_v.at[pl.ds(0, 32)], sg1)
    bg0.wait()
    bw0 = pltpu.async_copy(
        wbuf0_v.at[pl.ds(0, 32)], bertc_hbm.at[b, pl.ds(h * GHALF, 32)], sw0)
    bg1.wait()
    bw1 = pltpu.async_copy(
        wbuf1_v.at[pl.ds(0, 32)], bertc_hbm.at[b, pl.ds(h * GHALF + 32, 32)], sw1)
    gg.wait()
    gw = pltpu.async_copy(gbuf_v, gnnc_hbm.at[b, pl.ds(h * GHALF, GHALF)], sm)
    bw0.wait()
    bw1.wait()
    gw.wait()

    @pl.when((cid == 0) & (sid == 0))
    def _():
        pltpu.sync_copy(uid_hbm, uidx_v)
        pltpu.async_copy(user_hbm.at[uidx_v], ubuf_v, sm).wait()
        pltpu.sync_copy(ubuf_v, ue_hbm)


@functools.cache
def _make_sc_call():
    return pl.kernel(
        _sc_pack,
        out_type=[
            jax.ShapeDtypeStruct((B, LPAD, H), jnp.float32),
            jax.ShapeDtypeStruct((B, G, H), jnp.float32),
            jax.ShapeDtypeStruct((B, G, GH), jnp.float32),
            jax.ShapeDtypeStruct((B, GH), jnp.float32),
        ],
        mesh=plsc.VectorSubcoreMesh(core_axis_name="c", subcore_axis_name="s"),
        compiler_params=pltpu.CompilerParams(needs_layout_passes=False),
        scratch_types=[
            pltpu.VMEM((T,), jnp.int32),
            pltpu.VMEM((G,), jnp.int32),
            pltpu.VMEM((LPAD,), jnp.int32),
            pltpu.VMEM((G,), jnp.int32),
            pltpu.VMEM((KCH, H), jnp.float32),
            pltpu.VMEM((KCH, H), jnp.float32),
            pltpu.VMEM((KCH, H), jnp.float32),
            pltpu.VMEM((GHALF, GH), jnp.float32),
            pltpu.VMEM((B, GH), jnp.float32),
            pltpu.VMEM((B,), jnp.int32),
            pltpu.SemaphoreType.DMA,
            pltpu.SemaphoreType.DMA,
            pltpu.SemaphoreType.DMA,
            pltpu.SemaphoreType.DMA,
            pltpu.SemaphoreType.DMA,
            pltpu.SemaphoreType.DMA,
            pltpu.SemaphoreType.DMA,
        ],
    )


def _tc_finish(text_ref, tag_ref, base_ref, bertc_ref, gnnc_ref, ue_ref,
               pos_ref, wtag_ref, btag_ref, wuser_ref, buser_ref, type_ref,
               lnw_ref, lnb_ref, out_ref, attn_ref):
    bidx = pl.program_id(0)
    trow = text_ref[0]
    grow = tag_ref[0]
    text_cnt = jnp.sum((trow[:, 1:] > 0).astype(jnp.int32))
    tag_cnt = jnp.sum((grow > 0).astype(jnp.int32))
    text_end = text_cnt + 3
    tag_end = text_end + tag_cnt

    ge = lax.dot_general(gnnc_ref[0], wtag_ref[...], (((1,), (1,)), ((), ())),
                         preferred_element_type=jnp.float32)
    tage = (ge + btag_ref[...]) * bertc_ref[0]
    tage_pad = jnp.concatenate(
        [tage, jnp.zeros((LPAD - G, H), jnp.float32)], axis=0)
    emb = base_ref[0] + pltpu.roll(tage_pad, text_end, 0)

    pos_a = pos_ref[0:LPAD, :]
    prow = pos_ref[pl.ds(text_end, 1), :]
    shift = tag_cnt - 1
    shift = jnp.where(shift < 0, shift + LPAD, shift)
    pos_c = pltpu.roll(pos_a, shift, 0)
    g = lax.broadcasted_iota(jnp.int32, (LPAD, 1), 0)
    emb = emb + jnp.where(g < text_end, pos_a,
                          jnp.where(g < tag_end, prow, pos_c))

    ue_row = ue_ref[pl.ds(bidx, 1), :]
    uevec = lax.dot_general(ue_row, wuser_ref[...], (((1,), (1,)), ((), ())),
                            preferred_element_type=jnp.float32) + buser_ref[...]
    emb = emb + jnp.where(g == 1, 1.0, 0.0) * uevec

    t1 = ((g >= 3) & (g < text_end)).astype(jnp.float32)
    t2 = ((g >= text_end) & (g <= tag_end)).astype(jnp.float32)
    typ = type_ref[...]
    emb = (emb + typ[0:1] + t1 * (typ[1:2] - typ[0:1])
           + t2 * (typ[2:3] - typ[0:1]))

    mu = jnp.mean(emb, axis=-1, keepdims=True)
    var = jnp.mean((emb - mu) ** 2, axis=-1, keepdims=True)
    nrm = (emb - mu) * lax.rsqrt(var + 1e-12) * lnw_ref[...] + lnb_ref[...]
    out_ref[0] = nrm[:L]
    attn_ref[0] = (lax.broadcasted_iota(jnp.int32, (1, L), 1)
                   <= tag_end).astype(jnp.int32)


def kernel(user_ids, text_ids, tag_ids, user_table, word_table, bert_tag_table,
           gnn_tag_table, pos_table, type_table, W_user, b_user, W_tag, b_tag,
           ln_w, ln_b):
    text_ids = text_ids.astype(jnp.int32)
    tag_ids = tag_ids.astype(jnp.int32)
    uid_flat = user_ids.reshape(B).astype(jnp.int32)

    base, bert_c, gnn_c, ue = _make_sc_call()(
        text_ids, tag_ids, uid_flat, word_table, bert_tag_table,
        gnn_tag_table, user_table)

    text3 = text_ids.reshape(B, 1, T)
    tag3 = tag_ids.reshape(B, 1, G)
    out, attn = pl.pallas_call(
        _tc_finish,
        grid=(B,),
        in_specs=[
            pl.BlockSpec((1, 1, T), lambda i: (i, 0, 0)),
            pl.BlockSpec((1, 1, G), lambda i: (i, 0, 0)),
            pl.BlockSpec((1, LPAD, H), lambda i: (i, 0, 0)),
            pl.BlockSpec((1, G, H), lambda i: (i, 0, 0)),
            pl.BlockSpec((1, G, GH), lambda i: (i, 0, 0)),
            pl.BlockSpec((B, GH), lambda i: (0, 0)),
            pl.BlockSpec((1024, H), lambda i: (0, 0)),
            pl.BlockSpec((H, GH), lambda i: (0, 0)),
            pl.BlockSpec((1, H), lambda i: (0, 0)),
            pl.BlockSpec((H, GH), lambda i: (0, 0)),
            pl.BlockSpec((1, H), lambda i: (0, 0)),
            pl.BlockSpec((3, H), lambda i: (0, 0)),
            pl.BlockSpec((1, H), lambda i: (0, 0)),
            pl.BlockSpec((1, H), lambda i: (0, 0)),
        ],
        out_specs=[
            pl.BlockSpec((1, L, H), lambda i: (i, 0, 0)),
            pl.BlockSpec((1, 1, L), lambda i: (i, 0, 0)),
        ],
        out_shape=[
            jax.ShapeDtypeStruct((B, L, H), jnp.float32),
            jax.ShapeDtypeStruct((B, 1, L), jnp.int32),
        ],
    )(text3, tag3, base, bert_c, gnn_c, ue, pos_table, W_tag,
      b_tag.reshape(1, H), W_user, b_user.reshape(1, H), type_table,
      ln_w.reshape(1, H), ln_b.reshape(1, H))
    return out, attn.reshape(B, L)


# speedup vs baseline: 1.0749x; 1.0521x over previous
import functools

import jax
import jax.numpy as jnp
from jax import lax
from jax.experimental import pallas as pl
from jax.experimental.pallas import tpu as pltpu
from jax.experimental.pallas import tpu_sc as plsc

B, T, G = 16, 512, 128
H, GH = 768, 128
L = T + G + 3
LPAD = 672
LOUT = 648
HALF = 336
KCH = 48
NCH = 6
GHALF = G // 2


def _sc_pack(text_hbm, tag_hbm, uid_hbm, word_hbm, bert_hbm, gnn_hbm,
             user_hbm, base_hbm, bertc_hbm, gnnc_hbm, ue_hbm,
             tids_v, gids_v, wsrc_v, tcmp_v, wbuf0_v, wbuf1_v, wbuf2_v, gbuf_v,
             ubuf_v, uidx_v, sg0, sg1, sg2, sw0, sw1, sw2, sm):
    cid = lax.axis_index("c")
    sid = lax.axis_index("s")
    b = cid * 8 + sid // 2
    h = sid % 2
    lane = lax.iota(jnp.int32, 16)
    zero16 = jnp.zeros((16,), jnp.int32)

    pltpu.sync_copy(text_hbm.at[b], tids_v)
    pltpu.sync_copy(tag_hbm.at[b], gids_v)

    def zbody(i, c):
        wsrc_v[pl.ds(i * 16, 16)] = zero16
        return c
    lax.fori_loop(0, LPAD // 16, zbody, 0)

    def z2body(i, c):
        tcmp_v[pl.ds(i * 16, 16)] = zero16
        return c
    lax.fori_loop(0, G // 16, z2body, 0)

    def tbody(j, cnt):
        v = tids_v[pl.ds(j * 16, 16)]
        m = (v > 0) & ((lane + j * 16) > 0)
        mi = m.astype(jnp.int32)
        cs = plsc.cumsum(mi) + cnt
        plsc.store_scatter(wsrc_v, [cs + 2], v, mask=m)
        return cnt + jnp.sum(mi)
    text_cnt = lax.fori_loop(0, T // 16, tbody, jnp.int32(0))
    text_end = text_cnt + 3

    def gbody(j, cnt):
        v = gids_v[pl.ds(j * 16, 16)]
        m = v > 0
        mi = m.astype(jnp.int32)
        cs = plsc.cumsum(mi) + cnt
        plsc.store_scatter(tcmp_v, [cs - 1], v, mask=m)
        return cnt + jnp.sum(mi)
    tag_cnt = lax.fori_loop(0, G // 16, gbody, jnp.int32(0))
    tag_end = text_end + tag_cnt

    sep_col = jnp.where(text_cnt > 0, text_cnt, T - 1)
    sep_vec = plsc.load_gather(tids_v, [jnp.broadcast_to(sep_col, (16,))])
    cls_vec = plsc.load_gather(tids_v, [zero16])
    sp_idx = jnp.where(lane == 0, 0, jnp.where(lane == 1, 2, tag_end))
    sp_val = jnp.where(lane == 0, cls_vec, sep_vec)
    plsc.store_scatter(wsrc_v, [sp_idx], sp_val, mask=lane < 3)

    bufs = (wbuf0_v, wbuf1_v, wbuf2_v)
    gsems = (sg0, sg1, sg2)
    wsems = (sw0, sw1, sw2)
    NBUF = 3

    def g_start(j):
        c0 = h * HALF + j * KCH
        return pltpu.async_copy(
            word_hbm.at[wsrc_v.at[pl.ds(c0, KCH)]], bufs[j % NBUF], gsems[j % NBUF])

    def w_start(j):
        c0 = h * HALF + j * KCH
        return pltpu.async_copy(
            bufs[j % NBUF], base_hbm.at[b, pl.ds(c0, KCH)], wsems[j % NBUF])

    gd = {j: g_start(j) for j in range(NBUF)}
    wd = {}
    for j in range(NCH):
        gd[j].wait()
        wd[j] = w_start(j)
        if j + NBUF < NCH:
            wd[j].wait()
            gd[j + NBUF] = g_start(j + NBUF)

    gg = pltpu.async_copy(gnn_hbm.at[tcmp_v.at[pl.ds(h * GHALF, GHALF)]], gbuf_v, sm)
    wd[NCH - 3].wait()

    @pl.when(h == 0)
    def _():
        gt = pltpu.async_copy(
            word_hbm.at[wsrc_v.at[pl.ds(288, KCH)]], wbuf0_v, sg0)
        gt.wait()
        wt = pltpu.async_copy(wbuf0_v, base_hbm.at[b, pl.ds(288, KCH)], sw0)
        wt.wait()

    @pl.when(h == 1)
    def _():
        gt = pltpu.async_copy(
            word_hbm.at[wsrc_v.at[pl.ds(624, 24)]], wbuf0_v.at[pl.ds(0, 24)], sg0)
        gt.wait()
        wt = pltpu.async_copy(
            wbuf0_v.at[pl.ds(0, 24)], base_hbm.at[b, pl.ds(624, 24)], sw0)
        wt.wait()

    wd[NCH - 2].wait()
    wd[NCH - 1].wait()
    bg0 = pltpu.async_copy(
        bert_hbm.at[tcmp_v.at[pl.ds(h * GHALF, 32)]], wbuf0_v.at[pl.ds(0, 32)], sg0)
    bg1 = pltpu.async_copy(
        bert_hbm.at[tcmp_v.at[pl.ds(h * GHALF + 32, 32)]], wbuf1_v.at[pl.ds(0, 32)], sg1)
    bg0.wait()
    bw0 = pltpu.async_copy(
        wbuf0_v.at[pl.ds(0, 32)], bertc_hbm.at[b, pl.ds(h * GHALF, 32)], sw0)
    bg1.wait()
    bw1 = pltpu.async_copy(
        wbuf1_v.at[pl.ds(0, 32)], bertc_hbm.at[b, pl.ds(h * GHALF + 32, 32)], sw1)
    gg.wait()
    gw = pltpu.async_copy(gbuf_v, gnnc_hbm.at[b, pl.ds(h * GHALF, GHALF)], sm)
    bw0.wait()
    bw1.wait()
    gw.wait()

    @pl.when((cid == 0) & (sid == 0))
    def _():
        pltpu.sync_copy(uid_hbm, uidx_v)
        pltpu.async_copy(user_hbm.at[uidx_v], ubuf_v, sm).wait()
        pltpu.sync_copy(ubuf_v, ue_hbm)


@functools.cache
def _make_sc_call():
    return pl.kernel(
        _sc_pack,
        out_type=[
            jax.ShapeDtypeStruct((B, LOUT, H), jnp.float32),
            jax.ShapeDtypeStruct((B, G, H), jnp.float32),
            jax.ShapeDtypeStruct((B, G, GH), jnp.float32),
            jax.ShapeDtypeStruct((B, GH), jnp.float32),
        ],
        mesh=plsc.VectorSubcoreMesh(core_axis_name="c", subcore_axis_name="s"),
        compiler_params=pltpu.CompilerParams(needs_layout_passes=False),
        scratch_types=[
            pltpu.VMEM((T,), jnp.int32),
            pltpu.VMEM((G,), jnp.int32),
            pltpu.VMEM((LPAD,), jnp.int32),
            pltpu.VMEM((G,), jnp.int32),
            pltpu.VMEM((KCH, H), jnp.float32),
            pltpu.VMEM((KCH, H), jnp.float32),
            pltpu.VMEM((KCH, H), jnp.float32),
            pltpu.VMEM((GHALF, GH), jnp.float32),
            pltpu.VMEM((B, GH), jnp.float32),
            pltpu.VMEM((B,), jnp.int32),
            pltpu.SemaphoreType.DMA,
            pltpu.SemaphoreType.DMA,
            pltpu.SemaphoreType.DMA,
            pltpu.SemaphoreType.DMA,
            pltpu.SemaphoreType.DMA,
            pltpu.SemaphoreType.DMA,
            pltpu.SemaphoreType.DMA,
        ],
    )


def _tc_finish(text_ref, tag_ref, base_ref, bertc_ref, gnnc_ref, ue_ref,
               pos_ref, wtag_ref, btag_ref, wuser_ref, buser_ref, type_ref,
               lnw_ref, lnb_ref, out_ref, attn_ref):
    bidx = pl.program_id(0)
    trow = text_ref[0]
    grow = tag_ref[0]
    text_cnt = jnp.sum((trow[:, 1:] > 0).astype(jnp.int32))
    tag_cnt = jnp.sum((grow > 0).astype(jnp.int32))
    text_end = text_cnt + 3
    tag_end = text_end + tag_cnt

    ge = lax.dot_general(gnnc_ref[0], wtag_ref[...], (((1,), (1,)), ((), ())),
                         preferred_element_type=jnp.float32)
    tage = (ge + btag_ref[...]) * bertc_ref[0]
    tage_pad = jnp.concatenate(
        [tage, jnp.zeros((LOUT - G, H), jnp.float32)], axis=0)
    emb = base_ref[0] + pltpu.roll(tage_pad, text_end, 0)

    pos_a = pos_ref[0:LOUT, :]
    prow = pos_ref[pl.ds(text_end, 1), :]
    shift = tag_cnt - 1
    shift = jnp.where(shift < 0, shift + LOUT, shift)
    pos_c = pltpu.roll(pos_a, shift, 0)
    g = lax.broadcasted_iota(jnp.int32, (LOUT, 1), 0)
    emb = emb + jnp.where(g < text_end, pos_a,
                          jnp.where(g < tag_end, prow, pos_c))

    ue_row = ue_ref[pl.ds(bidx, 1), :]
    uevec = lax.dot_general(ue_row, wuser_ref[...], (((1,), (1,)), ((), ())),
                            preferred_element_type=jnp.float32) + buser_ref[...]
    emb = emb + jnp.where(g == 1, 1.0, 0.0) * uevec

    t1 = ((g >= 3) & (g < text_end)).astype(jnp.float32)
    t2 = ((g >= text_end) & (g <= tag_end)).astype(jnp.float32)
    typ = type_ref[...]
    emb = (emb + typ[0:1] + t1 * (typ[1:2] - typ[0:1])
           + t2 * (typ[2:3] - typ[0:1]))

    mu = jnp.mean(emb, axis=-1, keepdims=True)
    var = jnp.mean((emb - mu) ** 2, axis=-1, keepdims=True)
    nrm = (emb - mu) * lax.rsqrt(var + 1e-12) * lnw_ref[...] + lnb_ref[...]
    out_ref[0] = nrm[:L]
    attn_ref[0] = (lax.broadcasted_iota(jnp.int32, (1, L), 1)
                   <= tag_end).astype(jnp.int32)


def kernel(user_ids, text_ids, tag_ids, user_table, word_table, bert_tag_table,
           gnn_tag_table, pos_table, type_table, W_user, b_user, W_tag, b_tag,
           ln_w, ln_b):
    text_ids = text_ids.astype(jnp.int32)
    tag_ids = tag_ids.astype(jnp.int32)
    uid_flat = user_ids.reshape(B).astype(jnp.int32)

    base, bert_c, gnn_c, ue = _make_sc_call()(
        text_ids, tag_ids, uid_flat, word_table, bert_tag_table,
        gnn_tag_table, user_table)

    text3 = text_ids.reshape(B, 1, T)
    tag3 = tag_ids.reshape(B, 1, G)
    out, attn = pl.pallas_call(
        _tc_finish,
        grid=(B,),
        in_specs=[
            pl.BlockSpec((1, 1, T), lambda i: (i, 0, 0)),
            pl.BlockSpec((1, 1, G), lambda i: (i, 0, 0)),
            pl.BlockSpec((1, LOUT, H), lambda i: (i, 0, 0)),
            pl.BlockSpec((1, G, H), lambda i: (i, 0, 0)),
            pl.BlockSpec((1, G, GH), lambda i: (i, 0, 0)),
            pl.BlockSpec((B, GH), lambda i: (0, 0)),
            pl.BlockSpec((1024, H), lambda i: (0, 0)),
            pl.BlockSpec((H, GH), lambda i: (0, 0)),
            pl.BlockSpec((1, H), lambda i: (0, 0)),
            pl.BlockSpec((H, GH), lambda i: (0, 0)),
            pl.BlockSpec((1, H), lambda i: (0, 0)),
            pl.BlockSpec((3, H), lambda i: (0, 0)),
            pl.BlockSpec((1, H), lambda i: (0, 0)),
            pl.BlockSpec((1, H), lambda i: (0, 0)),
        ],
        out_specs=[
            pl.BlockSpec((1, L, H), lambda i: (i, 0, 0)),
            pl.BlockSpec((1, 1, L), lambda i: (i, 0, 0)),
        ],
        out_shape=[
            jax.ShapeDtypeStruct((B, L, H), jnp.float32),
            jax.ShapeDtypeStruct((B, 1, L), jnp.int32),
        ],
    )(text3, tag3, base, bert_c, gnn_c, ue, pos_table, W_tag,
      b_tag.reshape(1, H), W_user, b_user.reshape(1, H), type_table,
      ln_w.reshape(1, H), ln_b.reshape(1, H))
    return out, attn.reshape(B, L)
